# baseline (speedup 1.0000x reference)
import functools

import jax
import jax.numpy as jnp
from jax import lax
from jax.experimental import pallas as pl
from jax.experimental.pallas import tpu as pltpu
from jax.experimental.pallas import tpu_sc as plsc

B, HW = 8, 576
INPUT_DIM, HIDDEN, LATENT, NUM_EMBED = 64, 512, 256, 8192
OUT_DIM = LATENT + 2
BETA = 0.25
ROWS = B * HW

M_BLK = 512
M_GRID = ROWS // M_BLK
N_CHUNK = 1024
N_CHUNKS = NUM_EMBED // N_CHUNK

NW = 32
ROWS_PER_W = ROWS // NW
ROWS_PER_B = HW // ROWS_PER_W
GCHUNK = 72


def _decode_argmin_body(z_ref, w0, b0, w1, b1, w2, b2, w3, b3,
                        wl, bl, ws, bs_, cb_ref,
                        sr_ref, ids_ref, loss_ref, cbt_ref, s2_ref, acc_ref):
    step = pl.program_id(0)

    @pl.when(step == 0)
    def _():
        s2_ref[...] = jnp.sum(cb_ref[...] ** 2, axis=0, keepdims=True)

    tc = jnp.minimum(step, N_CHUNKS - 1)
    cbt_ref[...] = cb_ref[:, pl.ds(tc * N_CHUNK, N_CHUNK)].T

    x = z_ref[...]
    x = jnp.maximum(jnp.dot(x, w0[...], preferred_element_type=jnp.float32) + b0[...], 0.0)
    x = jnp.maximum(jnp.dot(x, w1[...], preferred_element_type=jnp.float32) + b1[...], 0.0)
    x = jnp.maximum(jnp.dot(x, w2[...], preferred_element_type=jnp.float32) + b2[...], 0.0)
    x = jnp.maximum(jnp.dot(x, w3[...], preferred_element_type=jnp.float32) + b3[...], 0.0)
    zl = jnp.dot(x, wl[...], preferred_element_type=jnp.float32) + bl[...]
    sr = jnp.dot(x, ws[...], preferred_element_type=jnp.float32) + bs_[...]

    @pl.when(step < B)
    def _():
        rsel = lax.broadcasted_iota(jnp.int32, (M_BLK, 2), 0) == step * 64
        sr_ref[...] = jnp.sum(jnp.where(rsel, sr, 0.0), axis=0, keepdims=True)[None]

    s1 = jnp.sum(zl ** 2, axis=1, keepdims=True)
    zl2 = zl + zl
    vmin = jnp.full((M_BLK, N_CHUNK), jnp.inf, dtype=jnp.float32)
    cidx = jnp.zeros((M_BLK, N_CHUNK), dtype=jnp.int32)
    for c in range(N_CHUNKS):
        cb_c = cb_ref[:, pl.ds(c * N_CHUNK, N_CHUNK)]
        s2 = s2_ref[:, pl.ds(c * N_CHUNK, N_CHUNK)]
        m2 = jnp.dot(zl2, cb_c, preferred_element_type=jnp.float32)
        d = (s1 + s2) - m2
        lt = d < vmin
        vmin = jnp.where(lt, d, vmin)
        cidx = jnp.where(lt, c, cidx)
    rowmin = jnp.min(vmin, axis=1)
    col = cidx * N_CHUNK + lax.broadcasted_iota(jnp.int32, (M_BLK, N_CHUNK), 1)
    cand = jnp.where(vmin == rowmin[:, None], col, jnp.int32(2 ** 30))
    ids_ref[...] = jnp.min(cand, axis=1)

    part = jnp.sum(rowmin)[None, None]
    acc = jnp.where(step == 0, part, acc_ref[...] + part)
    acc_ref[...] = acc

    @pl.when(step == M_GRID - 1)
    def _():
        msq = acc[0, 0] / jnp.float32(ROWS * LATENT)
        loss_ref[...] = (msq + msq * BETA)[None, None]


def _sc_gather_body(cbt_hbm, ids_hbm, out_hbm, idx_v, zq_v, gsem, osem):
    wid = lax.axis_index("s") * 2 + lax.axis_index("c")
    base = wid * ROWS_PER_W
    b = wid // ROWS_PER_B
    r0 = (wid % ROWS_PER_B) * ROWS_PER_W
    pltpu.sync_copy(ids_hbm.at[pl.ds(base, ROWS_PER_W)], idx_v)
    n_g = ROWS_PER_W // GCHUNK
    gathers = [
        pltpu.async_copy(
            cbt_hbm.at[idx_v.at[pl.ds(g * GCHUNK, GCHUNK)]],
            zq_v.at[pl.ds(g * GCHUNK, GCHUNK)], gsem)
        for g in range(n_g)
    ]
    stores = []
    for g in range(n_g):
        gathers[g].wait()
        stores.append(pltpu.async_copy(
            zq_v.at[pl.ds(g * GCHUNK, GCHUNK)],
            out_hbm.at[b, pl.ds(r0 + g * GCHUNK, GCHUNK)], osem))
    for st in stores:
        st.wait()


def _sc_gather(cbt, ids):
    run = functools.partial(
        pl.kernel,
        out_type=jax.ShapeDtypeStruct((B, HW, LATENT), jnp.float32),
        mesh=plsc.VectorSubcoreMesh(core_axis_name="c", subcore_axis_name="s",
                                    num_cores=2),
        scratch_types=[
            pltpu.VMEM((ROWS_PER_W,), jnp.int32),
            pltpu.VMEM((ROWS_PER_W, LATENT), jnp.float32),
            pltpu.SemaphoreType.DMA,
            pltpu.SemaphoreType.DMA,
        ],
    )(_sc_gather_body)
    return run(cbt, ids)


@jax.jit
def kernel(z, W0, b0, W1, b1, W2, b2, W3, b3, Wout, bout, codebook):
    zf = z.reshape(ROWS, INPUT_DIM)
    wl, ws = Wout[:, :LATENT], Wout[:, LATENT:]
    bl, bs_ = bout[:LATENT][None, :], bout[LATENT:][None, :]

    full = lambda shape: pl.BlockSpec(shape, lambda i: (0,) * len(shape))
    sr_out, ids, loss_out, cbt = pl.pallas_call(
        _decode_argmin_body,
        grid=(M_GRID,),
        in_specs=[
            pl.BlockSpec((M_BLK, INPUT_DIM), lambda i: (i, 0)),
            full((INPUT_DIM, HIDDEN)), full((1, HIDDEN)),
            full((HIDDEN, HIDDEN)), full((1, HIDDEN)),
            full((HIDDEN, HIDDEN)), full((1, HIDDEN)),
            full((HIDDEN, HIDDEN)), full((1, HIDDEN)),
            full((HIDDEN, LATENT)), full((1, LATENT)),
            full((HIDDEN, 2)), full((1, 2)),
            full((LATENT, NUM_EMBED)),
        ],
        out_specs=[
            pl.BlockSpec((1, 1, 2), lambda i: (jnp.minimum(i, B - 1), 0, 0)),
            pl.BlockSpec((M_BLK,), lambda i: (i,)),
            pl.BlockSpec((1, 1), lambda i: (0, 0)),
            pl.BlockSpec((N_CHUNK, LATENT),
                         lambda i: (jnp.minimum(i, N_CHUNKS - 1), 0)),
        ],
        out_shape=[
            jax.ShapeDtypeStruct((B, 1, 2), jnp.float32),
            jax.ShapeDtypeStruct((ROWS,), jnp.int32),
            jax.ShapeDtypeStruct((1, 1), jnp.float32),
            jax.ShapeDtypeStruct((NUM_EMBED, LATENT), jnp.float32),
        ],
        scratch_shapes=[pltpu.VMEM((1, NUM_EMBED), jnp.float32),
                        pltpu.VMEM((1, 1), jnp.float32)],
    )(zf, W0, b0[None, :], W1, b1[None, :], W2, b2[None, :], W3, b3[None, :],
      wl, bl, ws, bs_, codebook)

    zq_st = _sc_gather(cbt, ids)

    loss = loss_out.reshape(())
    scaler = sr_out[:, 0, 0]
    redshift = sr_out[:, 0, 1]
    return (zq_st, scaler, redshift, loss, ids)


# speedup vs baseline: 1.0037x; 1.0037x over previous
import functools

import jax
import jax.numpy as jnp
from jax import lax
from jax.experimental import pallas as pl
from jax.experimental.pallas import tpu as pltpu
from jax.experimental.pallas import tpu_sc as plsc

B, HW = 8, 576
INPUT_DIM, HIDDEN, LATENT, NUM_EMBED = 64, 512, 256, 8192
OUT_DIM = LATENT + 2
BETA = 0.25
ROWS = B * HW

M_BLK = 512
M_GRID = ROWS // M_BLK
N_CHUNK = 1024
N_CHUNKS = NUM_EMBED // N_CHUNK

NW = 32
ROWS_PER_W = ROWS // NW
ROWS_PER_B = HW // ROWS_PER_W
GCHUNK = 72


def _decode_argmin_body(z_ref, w0, b0, w1, b1, w2, b2, w3, b3,
                        wl, bl, ws, bs_, cb_ref,
                        sr_ref, ids_ref, loss_ref, cbt_ref, s2_ref, acc_ref):
    step = pl.program_id(0)

    @pl.when(step == 0)
    def _():
        s2_ref[...] = jnp.sum(cb_ref[...] ** 2, axis=0, keepdims=True)

    tc = jnp.minimum(step, N_CHUNKS - 1)
    cbt_ref[...] = cb_ref[:, pl.ds(tc * N_CHUNK, N_CHUNK)].T

    x = z_ref[...]
    x = jnp.maximum(jnp.dot(x, w0[...], preferred_element_type=jnp.float32) + b0[...], 0.0)
    x = jnp.maximum(jnp.dot(x, w1[...], preferred_element_type=jnp.float32) + b1[...], 0.0)
    x = jnp.maximum(jnp.dot(x, w2[...], preferred_element_type=jnp.float32) + b2[...], 0.0)
    x = jnp.maximum(jnp.dot(x, w3[...], preferred_element_type=jnp.float32) + b3[...], 0.0)
    zl = jnp.dot(x, wl[...], preferred_element_type=jnp.float32) + bl[...]
    sr = jnp.dot(x, ws[...], preferred_element_type=jnp.float32) + bs_[...]

    @pl.when(step < B)
    def _():
        rsel = lax.broadcasted_iota(jnp.int32, (M_BLK, 2), 0) == step * 64
        sr_ref[...] = jnp.sum(jnp.where(rsel, sr, 0.0), axis=0, keepdims=True)[None]

    s1 = jnp.sum(zl ** 2, axis=1, keepdims=True)
    zl2 = zl + zl
    vmin = jnp.full((M_BLK, N_CHUNK), jnp.inf, dtype=jnp.float32)
    cidx = jnp.zeros((M_BLK, N_CHUNK), dtype=jnp.int32)
    for c in range(N_CHUNKS):
        cb_c = cb_ref[:, pl.ds(c * N_CHUNK, N_CHUNK)]
        s2 = s2_ref[:, pl.ds(c * N_CHUNK, N_CHUNK)]
        m2 = jnp.dot(zl2, cb_c, preferred_element_type=jnp.float32)
        d = (s1 + s2) - m2
        lt = d < vmin
        vmin = jnp.where(lt, d, vmin)
        cidx = jnp.where(lt, c, cidx)
    rowmin = jnp.min(vmin, axis=1)
    col = cidx * N_CHUNK + lax.broadcasted_iota(jnp.int32, (M_BLK, N_CHUNK), 1)
    cand = jnp.where(vmin == rowmin[:, None], col, jnp.int32(2 ** 30))
    ids_ref[...] = jnp.min(cand, axis=1)

    part = jnp.sum(rowmin)[None, None]
    acc = jnp.where(step == 0, part, acc_ref[...] + part)
    acc_ref[...] = acc

    @pl.when(step == M_GRID - 1)
    def _():
        msq = acc[0, 0] / jnp.float32(ROWS * LATENT)
        loss_ref[...] = (msq + msq * BETA)[None, None]


def _sc_gather_body(cbt_hbm, ids_hbm, out_hbm, idx_v, zq_v, gsem):
    wid = lax.axis_index("s") * 2 + lax.axis_index("c")
    base = wid * ROWS_PER_W
    pltpu.sync_copy(ids_hbm.at[pl.ds(base, ROWS_PER_W)], idx_v)
    copies = [
        pltpu.async_copy(
            cbt_hbm.at[idx_v.at[pl.ds(g * GCHUNK, GCHUNK)]],
            zq_v.at[pl.ds(g * GCHUNK, GCHUNK)], gsem)
        for g in range(ROWS_PER_W // GCHUNK)
    ]
    for cp in copies:
        cp.wait()
    pltpu.sync_copy(zq_v, out_hbm.at[pl.ds(base, ROWS_PER_W)])


def _sc_gather(cbt, ids):
    run = functools.partial(
        pl.kernel,
        out_type=jax.ShapeDtypeStruct((ROWS, LATENT), jnp.float32),
        mesh=plsc.VectorSubcoreMesh(core_axis_name="c", subcore_axis_name="s",
                                    num_cores=2),
        scratch_types=[
            pltpu.VMEM((ROWS_PER_W,), jnp.int32),
            pltpu.VMEM((ROWS_PER_W, LATENT), jnp.float32),
            pltpu.SemaphoreType.DMA,
        ],
    )(_sc_gather_body)
    return run(cbt, ids)


@jax.jit
def kernel(z, W0, b0, W1, b1, W2, b2, W3, b3, Wout, bout, codebook):
    zf = z.reshape(ROWS, INPUT_DIM)
    wl, ws = Wout[:, :LATENT], Wout[:, LATENT:]
    bl, bs_ = bout[:LATENT][None, :], bout[LATENT:][None, :]

    full = lambda shape: pl.BlockSpec(shape, lambda i: (0,) * len(shape))
    sr_out, ids, loss_out, cbt = pl.pallas_call(
        _decode_argmin_body,
        grid=(M_GRID,),
        in_specs=[
            pl.BlockSpec((M_BLK, INPUT_DIM), lambda i: (i, 0)),
            full((INPUT_DIM, HIDDEN)), full((1, HIDDEN)),
            full((HIDDEN, HIDDEN)), full((1, HIDDEN)),
            full((HIDDEN, HIDDEN)), full((1, HIDDEN)),
            full((HIDDEN, HIDDEN)), full((1, HIDDEN)),
            full((HIDDEN, LATENT)), full((1, LATENT)),
            full((HIDDEN, 2)), full((1, 2)),
            full((LATENT, NUM_EMBED)),
        ],
        out_specs=[
            pl.BlockSpec((1, 1, 2), lambda i: (jnp.minimum(i, B - 1), 0, 0)),
            pl.BlockSpec((M_BLK,), lambda i: (i,)),
            pl.BlockSpec((1, 1), lambda i: (0, 0)),
            pl.BlockSpec((N_CHUNK, LATENT),
                         lambda i: (jnp.minimum(i, N_CHUNKS - 1), 0)),
        ],
        out_shape=[
            jax.ShapeDtypeStruct((B, 1, 2), jnp.float32),
            jax.ShapeDtypeStruct((ROWS,), jnp.int32),
            jax.ShapeDtypeStruct((1, 1), jnp.float32),
            jax.ShapeDtypeStruct((NUM_EMBED, LATENT), jnp.float32),
        ],
        scratch_shapes=[pltpu.VMEM((1, NUM_EMBED), jnp.float32),
                        pltpu.VMEM((1, 1), jnp.float32)],
    )(zf, W0, b0[None, :], W1, b1[None, :], W2, b2[None, :], W3, b3[None, :],
      wl, bl, ws, bs_, codebook)

    zq_st = _sc_gather(cbt, ids).reshape(B, HW, LATENT)

    loss = loss_out.reshape(())
    scaler = sr_out[:, 0, 0]
    redshift = sr_out[:, 0, 1]
    return (zq_st, scaler, redshift, loss, ids)


# speedup vs baseline: 1.0074x; 1.0037x over previous
import functools

import jax
import jax.numpy as jnp
from jax import lax
from jax.experimental import pallas as pl
from jax.experimental.pallas import tpu as pltpu
from jax.experimental.pallas import tpu_sc as plsc

B, HW = 8, 576
INPUT_DIM, HIDDEN, LATENT, NUM_EMBED = 64, 512, 256, 8192
OUT_DIM = LATENT + 2
BETA = 0.25
ROWS = B * HW

M_BLK = 512
N_CHUNK = 1024
N_CHUNKS = NUM_EMBED // N_CHUNK
ROWS_A = 2560
ROWS_B = ROWS - ROWS_A
TSTEP = 2 * N_CHUNK

NW = 32


def _make_tc_body(row0, emit_cbt):
    def body(z_ref, w0, b0, w1, b1, w2, b2, w3, b3,
             wl, bl, ws, bs_, cb_ref, *refs):
        if emit_cbt:
            sr_ref, ids_ref, acc_out, cbt_ref, s2_ref, acc_ref = refs
        else:
            sr_ref, ids_ref, acc_out, s2_ref, acc_ref = refs
        step = pl.program_id(0)
        nsteps = pl.num_programs(0)

        @pl.when(step == 0)
        def _():
            s2_ref[...] = jnp.sum(cb_ref[...] ** 2, axis=0, keepdims=True)

        if emit_cbt:
            tc = jnp.minimum(step, N_CHUNKS // 2 - 1)
            cbt_ref[...] = cb_ref[:, pl.ds(tc * TSTEP, TSTEP)].T

        x = z_ref[...]
        x = jnp.maximum(jnp.dot(x, w0[...], preferred_element_type=jnp.float32) + b0[...], 0.0)
        x = jnp.maximum(jnp.dot(x, w1[...], preferred_element_type=jnp.float32) + b1[...], 0.0)
        x = jnp.maximum(jnp.dot(x, w2[...], preferred_element_type=jnp.float32) + b2[...], 0.0)
        x = jnp.maximum(jnp.dot(x, w3[...], preferred_element_type=jnp.float32) + b3[...], 0.0)
        zl = jnp.dot(x, wl[...], preferred_element_type=jnp.float32) + bl[...]
        sr = jnp.dot(x, ws[...], preferred_element_type=jnp.float32) + bs_[...]

        start = row0 + step * M_BLK
        bq = (start + (HW - 1)) // HW
        local = bq * HW - start

        @pl.when((local < M_BLK) & (bq < B))
        def _():
            rsel = lax.broadcasted_iota(jnp.int32, (M_BLK, 2), 0) == local
            sr_ref[...] = jnp.sum(jnp.where(rsel, sr, 0.0), axis=0,
                                  keepdims=True)[None]

        s1 = jnp.sum(zl ** 2, axis=1, keepdims=True)
        zl2 = zl + zl
        vmin = jnp.full((M_BLK, N_CHUNK), jnp.inf, dtype=jnp.float32)
        cidx = jnp.zeros((M_BLK, N_CHUNK), dtype=jnp.int32)
        for c in range(N_CHUNKS):
            cb_c = cb_ref[:, pl.ds(c * N_CHUNK, N_CHUNK)]
            s2 = s2_ref[:, pl.ds(c * N_CHUNK, N_CHUNK)]
            m2 = jnp.dot(zl2, cb_c, preferred_element_type=jnp.float32)
            d = (s1 + s2) - m2
            lt = d < vmin
            vmin = jnp.where(lt, d, vmin)
            cidx = jnp.where(lt, c, cidx)
        rowmin = jnp.min(vmin, axis=1)
        col = cidx * N_CHUNK + lax.broadcasted_iota(jnp.int32, (M_BLK, N_CHUNK), 1)
        cand = jnp.where(vmin == rowmin[:, None], col, jnp.int32(2 ** 30))
        ids_ref[...] = jnp.min(cand, axis=1)

        part = jnp.sum(rowmin)[None, None]
        acc = jnp.where(step == 0, part, acc_ref[...] + part)
        acc_ref[...] = acc

        @pl.when(step == nsteps - 1)
        def _():
            acc_out[...] = acc

    return body


def _tc_call(args, row0, rows, emit_cbt):
    nsteps = rows // M_BLK
    full = lambda shape: pl.BlockSpec(shape, lambda i: (0,) * len(shape))

    def sr_idx(i):
        return (jnp.clip((row0 + i * M_BLK + (HW - 1)) // HW, 0, B - 1), 0, 0)

    out_specs = [
        pl.BlockSpec((1, 1, 2), sr_idx),
        pl.BlockSpec((M_BLK,), lambda i: (i,)),
        pl.BlockSpec((1, 1), lambda i: (0, 0)),
    ]
    out_shape = [
        jax.ShapeDtypeStruct((B, 1, 2), jnp.float32),
        jax.ShapeDtypeStruct((rows,), jnp.int32),
        jax.ShapeDtypeStruct((1, 1), jnp.float32),
    ]
    if emit_cbt:
        out_specs.append(pl.BlockSpec(
            (TSTEP, LATENT), lambda i: (jnp.minimum(i, N_CHUNKS // 2 - 1), 0)))
        out_shape.append(jax.ShapeDtypeStruct((NUM_EMBED, LATENT), jnp.float32))

    return pl.pallas_call(
        _make_tc_body(row0, emit_cbt),
        grid=(nsteps,),
        in_specs=[
            pl.BlockSpec((M_BLK, INPUT_DIM),
                         lambda i, row0=row0: (row0 // M_BLK + i, 0)),
            full((INPUT_DIM, HIDDEN)), full((1, HIDDEN)),
            full((HIDDEN, HIDDEN)), full((1, HIDDEN)),
            full((HIDDEN, HIDDEN)), full((1, HIDDEN)),
            full((HIDDEN, HIDDEN)), full((1, HIDDEN)),
            full((HIDDEN, LATENT)), full((1, LATENT)),
            full((HIDDEN, 2)), full((1, 2)),
            full((LATENT, NUM_EMBED)),
        ],
        out_specs=out_specs,
        out_shape=out_shape,
        scratch_shapes=[pltpu.VMEM((1, NUM_EMBED), jnp.float32),
                        pltpu.VMEM((1, 1), jnp.float32)],
    )(*args)


def _sc_gather_body(rows_per_w, cbt_hbm, ids_hbm, out_hbm, idx_v, zq_v, gsem):
    wid = lax.axis_index("s") * 2 + lax.axis_index("c")
    base = wid * rows_per_w
    pltpu.sync_copy(ids_hbm.at[pl.ds(base, rows_per_w)], idx_v)
    pltpu.async_copy(cbt_hbm.at[idx_v], zq_v, gsem).wait()
    pltpu.sync_copy(zq_v, out_hbm.at[pl.ds(base, rows_per_w)])


def _sc_gather(cbt, ids, rows):
    rows_per_w = rows // NW
    run = functools.partial(
        pl.kernel,
        out_type=jax.ShapeDtypeStruct((rows, LATENT), jnp.float32),
        mesh=plsc.VectorSubcoreMesh(core_axis_name="c", subcore_axis_name="s",
                                    num_cores=2),
        scratch_types=[
            pltpu.VMEM((rows_per_w,), jnp.int32),
            pltpu.VMEM((rows_per_w, LATENT), jnp.float32),
            pltpu.SemaphoreType.DMA,
        ],
    )(functools.partial(_sc_gather_body, rows_per_w))
    return run(cbt, ids)


@jax.jit
def kernel(z, W0, b0, W1, b1, W2, b2, W3, b3, Wout, bout, codebook):
    zf = z.reshape(ROWS, INPUT_DIM)
    wl, ws = Wout[:, :LATENT], Wout[:, LATENT:]
    bl, bs_ = bout[:LATENT][None, :], bout[LATENT:][None, :]
    args = (zf, W0, b0[None, :], W1, b1[None, :], W2, b2[None, :],
            W3, b3[None, :], wl, bl, ws, bs_, codebook)

    sr_a, ids_a, acc_a, cbt = _tc_call(args, 0, ROWS_A, True)
    zq_a = _sc_gather(cbt, ids_a, ROWS_A)
    sr_b, ids_b, acc_b = _tc_call(args, ROWS_A, ROWS_B, False)
    zq_b = _sc_gather(cbt, ids_b, ROWS_B)

    ids = jnp.concatenate([ids_a, ids_b])
    zq_st = jnp.concatenate([zq_a, zq_b]).reshape(B, HW, LATENT)

    msq = (acc_a[0, 0] + acc_b[0, 0]) / jnp.float32(ROWS * LATENT)
    loss = msq + msq * BETA

    nb_a = (ROWS_A + HW - 1) // HW
    scaler = jnp.concatenate([sr_a[:nb_a, 0, 0], sr_b[nb_a:, 0, 0]])
    redshift = jnp.concatenate([sr_a[:nb_a, 0, 1], sr_b[nb_a:, 0, 1]])
    return (zq_st, scaler, redshift, loss, ids)


# speedup vs baseline: 1.0195x; 1.0120x over previous
import functools

import jax
import jax.numpy as jnp
from jax import lax
from jax.experimental import pallas as pl
from jax.experimental.pallas import tpu as pltpu
from jax.experimental.pallas import tpu_sc as plsc

B, HW = 8, 576
INPUT_DIM, HIDDEN, LATENT, NUM_EMBED = 64, 512, 256, 8192
OUT_DIM = LATENT + 2
BETA = 0.25
ROWS = B * HW

M_BLK = 1152
M_GRID = ROWS // M_BLK
N_CHUNK = 1024
N_CHUNKS = NUM_EMBED // N_CHUNK
TSTEP = NUM_EMBED // M_GRID

NW = 32
ROWS_PER_W = ROWS // NW
GCHUNK = 72


def _decode_argmin_body(z_ref, w0, b0, w1, b1, w2, b2, w3, b3,
                        wl, bl, ws, bs_, cb_ref,
                        sr_ref, ids_ref, loss_ref, cbt_ref, s2_ref, acc_ref):
    step = pl.program_id(0)

    @pl.when(step == 0)
    def _():
        s2_ref[...] = jnp.sum(cb_ref[...] ** 2, axis=0, keepdims=True)

    cbt_ref[...] = cb_ref[:, pl.ds(step * TSTEP, TSTEP)].T

    x = z_ref[...]
    x = jnp.maximum(jnp.dot(x, w0[...], preferred_element_type=jnp.float32) + b0[...], 0.0)
    x = jnp.maximum(jnp.dot(x, w1[...], preferred_element_type=jnp.float32) + b1[...], 0.0)
    x = jnp.maximum(jnp.dot(x, w2[...], preferred_element_type=jnp.float32) + b2[...], 0.0)
    x = jnp.maximum(jnp.dot(x, w3[...], preferred_element_type=jnp.float32) + b3[...], 0.0)
    zl = jnp.dot(x, wl[...], preferred_element_type=jnp.float32) + bl[...]
    sr = jnp.dot(x, ws[...], preferred_element_type=jnp.float32) + bs_[...]

    riota = lax.broadcasted_iota(jnp.int32, (M_BLK, 2), 0)
    row_a = jnp.sum(jnp.where(riota == 0, sr, 0.0), axis=0, keepdims=True)
    row_b = jnp.sum(jnp.where(riota == HW, sr, 0.0), axis=0, keepdims=True)
    sr_ref[...] = jnp.concatenate([row_a, row_b], axis=0)[:, None, :]

    s1 = jnp.sum(zl ** 2, axis=1, keepdims=True)
    zl2 = zl + zl
    vmin = jnp.full((M_BLK, N_CHUNK), jnp.inf, dtype=jnp.float32)
    cidx = jnp.zeros((M_BLK, N_CHUNK), dtype=jnp.int32)
    for c in range(N_CHUNKS):
        cb_c = cb_ref[:, pl.ds(c * N_CHUNK, N_CHUNK)]
        s2 = s2_ref[:, pl.ds(c * N_CHUNK, N_CHUNK)]
        m2 = jnp.dot(zl2, cb_c, preferred_element_type=jnp.float32)
        d = (s1 + s2) - m2
        lt = d < vmin
        vmin = jnp.where(lt, d, vmin)
        cidx = jnp.where(lt, c, cidx)
    rowmin = jnp.min(vmin, axis=1)
    col = cidx * N_CHUNK + lax.broadcasted_iota(jnp.int32, (M_BLK, N_CHUNK), 1)
    cand = jnp.where(vmin == rowmin[:, None], col, jnp.int32(2 ** 30))
    ids_ref[...] = jnp.min(cand, axis=1)[None, None, :]

    part = jnp.sum(rowmin)[None, None]
    acc = jnp.where(step == 0, part, acc_ref[...] + part)
    acc_ref[...] = acc

    @pl.when(step == M_GRID - 1)
    def _():
        msq = acc[0, 0] / jnp.float32(ROWS * LATENT)
        loss_ref[...] = (msq + msq * BETA)[None, None]


def _sc_gather_body(cbt_hbm, ids_hbm, out_hbm, idx_v, zq_v, gsem):
    wid = lax.axis_index("s") * 2 + lax.axis_index("c")
    base = wid * ROWS_PER_W
    pltpu.sync_copy(ids_hbm.at[pl.ds(base, ROWS_PER_W)], idx_v)
    copies = [
        pltpu.async_copy(
            cbt_hbm.at[idx_v.at[pl.ds(g * GCHUNK, GCHUNK)]],
            zq_v.at[pl.ds(g * GCHUNK, GCHUNK)], gsem)
        for g in range(ROWS_PER_W // GCHUNK)
    ]
    for cp in copies:
        cp.wait()
    pltpu.sync_copy(zq_v, out_hbm.at[pl.ds(base, ROWS_PER_W)])


def _sc_gather(cbt, ids):
    run = functools.partial(
        pl.kernel,
        out_type=jax.ShapeDtypeStruct((ROWS, LATENT), jnp.float32),
        mesh=plsc.VectorSubcoreMesh(core_axis_name="c", subcore_axis_name="s",
                                    num_cores=2),
        scratch_types=[
            pltpu.VMEM((ROWS_PER_W,), jnp.int32),
            pltpu.VMEM((ROWS_PER_W, LATENT), jnp.float32),
            pltpu.SemaphoreType.DMA,
        ],
    )(_sc_gather_body)
    return run(cbt, ids)


@jax.jit
def kernel(z, W0, b0, W1, b1, W2, b2, W3, b3, Wout, bout, codebook):
    zf = z.reshape(ROWS, INPUT_DIM)
    wl, ws = Wout[:, :LATENT], Wout[:, LATENT:]
    bl, bs_ = bout[:LATENT][None, :], bout[LATENT:][None, :]

    full = lambda shape: pl.BlockSpec(shape, lambda i: (0,) * len(shape))
    sr_out, ids_out, loss_out, cbt = pl.pallas_call(
        _decode_argmin_body,
        grid=(M_GRID,),
        in_specs=[
            pl.BlockSpec((M_BLK, INPUT_DIM), lambda i: (i, 0)),
            full((INPUT_DIM, HIDDEN)), full((1, HIDDEN)),
            full((HIDDEN, HIDDEN)), full((1, HIDDEN)),
            full((HIDDEN, HIDDEN)), full((1, HIDDEN)),
            full((HIDDEN, HIDDEN)), full((1, HIDDEN)),
            full((HIDDEN, LATENT)), full((1, LATENT)),
            full((HIDDEN, 2)), full((1, 2)),
            full((LATENT, NUM_EMBED)),
        ],
        out_specs=[
            pl.BlockSpec((2, 1, 2), lambda i: (i, 0, 0)),
            pl.BlockSpec((1, 1, M_BLK), lambda i: (i, 0, 0)),
            pl.BlockSpec((1, 1), lambda i: (0, 0)),
            pl.BlockSpec((TSTEP, LATENT), lambda i: (i, 0)),
        ],
        out_shape=[
            jax.ShapeDtypeStruct((B, 1, 2), jnp.float32),
            jax.ShapeDtypeStruct((M_GRID, 1, M_BLK), jnp.int32),
            jax.ShapeDtypeStruct((1, 1), jnp.float32),
            jax.ShapeDtypeStruct((NUM_EMBED, LATENT), jnp.float32),
        ],
        scratch_shapes=[pltpu.VMEM((1, NUM_EMBED), jnp.float32),
                        pltpu.VMEM((1, 1), jnp.float32)],
    )(zf, W0, b0[None, :], W1, b1[None, :], W2, b2[None, :], W3, b3[None, :],
      wl, bl, ws, bs_, codebook)

    ids = ids_out.reshape(ROWS)
    zq_st = _sc_gather(cbt, ids).reshape(B, HW, LATENT)

    loss = loss_out.reshape(())
    scaler = sr_out[:, 0, 0]
    redshift = sr_out[:, 0, 1]
    return (zq_st, scaler, redshift, loss, ids)
